# Initial kernel scaffold; baseline (speedup 1.0000x reference)
import functools

import jax
import jax.numpy as jnp
from jax import lax
from jax.experimental import pallas as pl
from jax.experimental.pallas import tpu as pltpu
from jax.experimental.pallas import tpu_sc as plsc

D = 64
CH = 128

_info = plsc.get_sparse_core_info()
NC = _info.num_cores
NS = _info.num_subcores
NW = NC * NS


def _make_emb(n_chunks):
  mesh = plsc.VectorSubcoreMesh(core_axis_name="c", subcore_axis_name="s")

  @functools.partial(
      pl.kernel,
      mesh=mesh,
      out_type=jax.ShapeDtypeStruct((NW, n_chunks, CH, D), jnp.float32),
      scratch_types=[
          pltpu.VMEM((n_chunks, CH), jnp.int32),
          pltpu.VMEM((CH, D), jnp.float32),
          pltpu.SemaphoreType.DMA,
      ],
  )
  def emb(idx_hbm, table_hbm, out_hbm, idx_v, rows_v, sem):
    wid = lax.axis_index("s") * NC + lax.axis_index("c")
    pltpu.sync_copy(idx_hbm.at[wid], idx_v)

    def body(j, carry):
      pltpu.async_copy(table_hbm.at[idx_v.at[j]], rows_v, sem).wait()
      pltpu.sync_copy(rows_v, out_hbm.at[wid].at[j])
      return carry

    lax.fori_loop(0, n_chunks, body, 0)

  return emb


def kernel(tokenized_sentence, embedding_table):
  b, s = tokenized_sentence.shape
  total = b * s
  n_chunks = total // (NW * CH)
  ids = tokenized_sentence.reshape(NW, n_chunks, CH).astype(jnp.int32)
  out = _make_emb(n_chunks)(ids, embedding_table)
  return out.reshape(b, s, D)


# speedup vs baseline: 1.6857x; 1.6857x over previous
import functools

import jax
import jax.numpy as jnp
from jax import lax
from jax.experimental import pallas as pl
from jax.experimental.pallas import tpu as pltpu
from jax.experimental.pallas import tpu_sc as plsc

D = 64
CH = 128

_info = plsc.get_sparse_core_info()
NC = _info.num_cores
NS = _info.num_subcores
NW = NC * NS


def _make_emb(n_chunks):
  mesh = plsc.VectorSubcoreMesh(core_axis_name="c", subcore_axis_name="s")

  @functools.partial(
      pl.kernel,
      mesh=mesh,
      compiler_params=pltpu.CompilerParams(use_tc_tiling_on_sc=False),
      out_type=jax.ShapeDtypeStruct((NW, n_chunks, CH, D), jnp.float32),
      scratch_types=[
          pltpu.VMEM((n_chunks, CH), jnp.int32),
          pltpu.VMEM((CH, D), jnp.float32),
          pltpu.SemaphoreType.DMA,
      ],
  )
  def emb(idx_hbm, table_hbm, out_hbm, idx_v, rows_v, sem):
    wid = lax.axis_index("s") * NC + lax.axis_index("c")
    pltpu.sync_copy(idx_hbm.at[wid], idx_v)

    def body(j, carry):
      pltpu.async_copy(table_hbm.at[idx_v.at[j]], rows_v, sem).wait()
      pltpu.sync_copy(rows_v, out_hbm.at[wid].at[j])
      return carry

    lax.fori_loop(0, n_chunks, body, 0)

  return emb


def kernel(tokenized_sentence, embedding_table):
  b, s = tokenized_sentence.shape
  total = b * s
  n_chunks = total // (NW * CH)
  ids = tokenized_sentence.reshape(NW, n_chunks, CH).astype(jnp.int32)
  out = _make_emb(n_chunks)(ids, embedding_table)
  return out.reshape(b, s, D)


# speedup vs baseline: 1.8746x; 1.1121x over previous
import functools

import jax
import jax.numpy as jnp
from jax import lax
from jax.experimental import pallas as pl
from jax.experimental.pallas import tpu as pltpu
from jax.experimental.pallas import tpu_sc as plsc

D = 64
CH = 128
NB = 8
K = 4

_info = plsc.get_sparse_core_info()
NC = _info.num_cores
NS = _info.num_subcores
NW = NC * NS


def _make_emb(n_chunks):
  mesh = plsc.VectorSubcoreMesh(core_axis_name="c", subcore_axis_name="s")
  n_rounds = n_chunks // NB

  @functools.partial(
      pl.kernel,
      mesh=mesh,
      compiler_params=pltpu.CompilerParams(use_tc_tiling_on_sc=False),
      out_type=jax.ShapeDtypeStruct((NW, n_chunks, CH, D), jnp.float32),
      scratch_types=[
          pltpu.VMEM((n_chunks, CH), jnp.int32),
          pltpu.VMEM((NB, CH, D), jnp.float32),
          pltpu.SemaphoreType.DMA((NB,)),
          pltpu.SemaphoreType.DMA((NB,)),
      ],
  )
  def emb(idx_hbm, table_hbm, out_hbm, idx_v, rows_v, gsem, ssem):
    wid = lax.axis_index("s") * NC + lax.axis_index("c")
    pltpu.sync_copy(idx_hbm.at[wid], idx_v)
    out_w = out_hbm.at[wid]

    def start_gather(j, b):
      pltpu.async_copy(table_hbm.at[idx_v.at[j]], rows_v.at[b], gsem.at[b])

    def wait_gather(b):
      pltpu.make_async_copy(
          table_hbm.at[idx_v.at[0]], rows_v.at[b], gsem.at[b]).wait()

    def start_store(j, b):
      pltpu.async_copy(rows_v.at[b], out_w.at[j], ssem.at[b])

    def wait_store(b):
      pltpu.make_async_copy(rows_v.at[b], out_w.at[0], ssem.at[b]).wait()

    for b in range(K):
      start_gather(b, b)

    for b in range(NB):
      wait_gather(b)
      start_store(b, b)
      f = b + K
      if f < NB:
        start_gather(f, f)
      else:
        bf = f % NB
        wait_store(bf)
        start_gather(f, bf)

    def round_body(r, carry):
      g = r * NB
      for b in range(NB):
        j = g + b
        wait_gather(b)
        start_store(j, b)
        bf = (b + K) % NB
        wait_store(bf)
        start_gather(j + K, bf)
      return carry

    lax.fori_loop(1, n_rounds - 1, round_body, 0)

    g = (n_rounds - 1) * NB
    for b in range(NB):
      j = g + b
      wait_gather(b)
      start_store(j, b)
      f = j + K
      if f < n_chunks:
        bf = (b + K) % NB
        wait_store(bf)
        start_gather(f, bf)
    for b in range(NB):
      wait_store(b)

  return emb


def kernel(tokenized_sentence, embedding_table):
  b, s = tokenized_sentence.shape
  total = b * s
  n_chunks = total // (NW * CH)
  ids = tokenized_sentence.reshape(NW, n_chunks, CH).astype(jnp.int32)
  out = _make_emb(n_chunks)(ids, embedding_table)
  return out.reshape(b, s, D)


# speedup vs baseline: 1.8896x; 1.0080x over previous
import functools

import jax
import jax.numpy as jnp
from jax import lax
from jax.experimental import pallas as pl
from jax.experimental.pallas import tpu as pltpu
from jax.experimental.pallas import tpu_sc as plsc

D = 64
CH = 256
NB = 4
K = 2

_info = plsc.get_sparse_core_info()
NC = _info.num_cores
NS = _info.num_subcores
NW = NC * NS


def _make_emb(n_chunks):
  mesh = plsc.VectorSubcoreMesh(core_axis_name="c", subcore_axis_name="s")
  n_rounds = n_chunks // NB

  @functools.partial(
      pl.kernel,
      mesh=mesh,
      compiler_params=pltpu.CompilerParams(use_tc_tiling_on_sc=False),
      out_type=jax.ShapeDtypeStruct((NW, n_chunks, CH, D), jnp.float32),
      scratch_types=[
          pltpu.VMEM((n_chunks, CH), jnp.int32),
          pltpu.VMEM((NB, CH, D), jnp.float32),
          pltpu.SemaphoreType.DMA((NB,)),
          pltpu.SemaphoreType.DMA((NB,)),
      ],
  )
  def emb(idx_hbm, table_hbm, out_hbm, idx_v, rows_v, gsem, ssem):
    wid = lax.axis_index("s") * NC + lax.axis_index("c")
    pltpu.sync_copy(idx_hbm.at[wid], idx_v)
    out_w = out_hbm.at[wid]

    def start_gather(j, b):
      pltpu.async_copy(table_hbm.at[idx_v.at[j]], rows_v.at[b], gsem.at[b])

    def wait_gather(b):
      pltpu.make_async_copy(
          table_hbm.at[idx_v.at[0]], rows_v.at[b], gsem.at[b]).wait()

    def start_store(j, b):
      pltpu.async_copy(rows_v.at[b], out_w.at[j], ssem.at[b])

    def wait_store(b):
      pltpu.make_async_copy(rows_v.at[b], out_w.at[0], ssem.at[b]).wait()

    for b in range(K):
      start_gather(b, b)

    for b in range(NB):
      wait_gather(b)
      start_store(b, b)
      f = b + K
      if f < NB:
        start_gather(f, f)
      else:
        bf = f % NB
        wait_store(bf)
        start_gather(f, bf)

    def round_body(r, carry):
      g = r * NB
      for b in range(NB):
        j = g + b
        wait_gather(b)
        start_store(j, b)
        bf = (b + K) % NB
        wait_store(bf)
        start_gather(j + K, bf)
      return carry

    lax.fori_loop(1, n_rounds - 1, round_body, 0)

    g = (n_rounds - 1) * NB
    for b in range(NB):
      j = g + b
      wait_gather(b)
      start_store(j, b)
      f = j + K
      if f < n_chunks:
        bf = (b + K) % NB
        wait_store(bf)
        start_gather(f, bf)
    for b in range(NB):
      wait_store(b)

  return emb


def kernel(tokenized_sentence, embedding_table):
  b, s = tokenized_sentence.shape
  total = b * s
  n_chunks = total // (NW * CH)
  ids = tokenized_sentence.reshape(NW, n_chunks, CH).astype(jnp.int32)
  out = _make_emb(n_chunks)(ids, embedding_table)
  return out.reshape(b, s, D)
